# baseline (speedup 1.0000x reference)
import jax
import jax.numpy as jnp
from jax import lax
from jax.experimental import pallas as pl

N_AGENTS = 8
N_ACTIONS = 16
STATE_DIM = 256
EMBED = 64
SAMPLE = 16
BLK = 256

_DN_W_X = (((0,), (1,)), ((), ()))
_DN_W_TX = (((0,), (0,)), ((), ()))


def _mixer_kernel(s_ref, ra_ref, wq_ref, aq_ref,
                  hw1_w1_ref, hw1_b1_ref, hw1_w2_ref, hw1_b2_ref,
                  hwf_w1_ref, hwf_b1_ref, hwf_w2_ref, hwf_b2_ref,
                  hb1_w_ref, hb1_b_ref, v_w1_ref, v_b1_ref, v_w2_ref, v_b2_ref,
                  west_ref, qtot_ref):
    f32 = jnp.float32
    s = s_ref[...]
    h1T = jnp.maximum(
        lax.dot_general(hw1_w1_ref[...], s, _DN_W_X,
                        preferred_element_type=f32) + hw1_b1_ref[...], 0.0)
    w1T = jnp.abs(
        lax.dot_general(hw1_w2_ref[...], h1T, _DN_W_TX,
                        preferred_element_type=f32) + hw1_b2_ref[...])
    hfT = jnp.maximum(
        lax.dot_general(hwf_w1_ref[...], s, _DN_W_X,
                        preferred_element_type=f32) + hwf_b1_ref[...], 0.0)
    wfT = jnp.abs(
        lax.dot_general(hwf_w2_ref[...], hfT, _DN_W_TX,
                        preferred_element_type=f32) + hwf_b2_ref[...])
    b1T = lax.dot_general(hb1_w_ref[...], s, _DN_W_X,
                          preferred_element_type=f32) + hb1_b_ref[...]
    hvT = jnp.maximum(
        lax.dot_general(v_w1_ref[...], s, _DN_W_X,
                        preferred_element_type=f32) + v_b1_ref[...], 0.0)
    vT = jnp.sum(hvT * v_w2_ref[...], axis=0, keepdims=True) + v_b2_ref[...]

    raT = ra_ref[...].T
    Wq = wq_ref[...]
    R = raT.shape[1]
    acnvT = jnp.zeros((N_AGENTS, N_ACTIONS, R), f32)
    for q in range(N_AGENTS):
        wqi = Wq[q * N_AGENTS:(q + 1) * N_AGENTS, :]
        raq = raT[q * N_ACTIONS:(q + 1) * N_ACTIONS, :]
        acnvT = acnvT + wqi[:, None, :] * raq[None, :, :]
    acnvT = acnvT.reshape(N_AGENTS * N_ACTIONS, R)

    rows = []
    for i in range(N_AGENTS):
        acc = b1T
        for c in range(N_ACTIONS):
            m = jnp.broadcast_to(acnvT[i * N_ACTIONS + c:i * N_ACTIONS + c + 1, :], (EMBED, R))
            acc = acc + m * w1T[c * EMBED:(c + 1) * EMBED, :]
        for c in range(N_ACTIONS):
            cc = N_ACTIONS + c
            m = jnp.broadcast_to(raT[i * N_ACTIONS + c:i * N_ACTIONS + c + 1, :], (EMBED, R))
            acc = acc + m * w1T[cc * EMBED:(cc + 1) * EMBED, :]
        hid = jnp.where(acc > 0, acc, jnp.exp(jnp.minimum(acc, 0.0)) - 1.0)
        y_i = jnp.sum(hid * wfT, axis=0, keepdims=True) + vT
        rows.append(jnp.abs(y_i))
    westT = jnp.concatenate(rows, axis=0)
    west_ref[...] = westT.T
    qtot = jnp.sum(westT * aq_ref[...].T, axis=0, keepdims=True)
    qtot_ref[...] = qtot.T


def _coalition_weights(bs):
    with jax.ensure_compile_time_eval():
        perm = jnp.argsort(
            jax.random.uniform(jax.random.key(42), (bs * SAMPLE, N_AGENTS)), axis=-1)
        perm3 = perm.reshape(bs, SAMPLE, N_AGENTS)
        inv = jnp.argsort(perm3, axis=-1)
        mask = (inv[:, :, None, :] < perm3[:, :, :, None]).astype(jnp.float32)
        W = (perm3[:, :, :, None].astype(jnp.float32) * mask).sum(axis=1)
        W = W / (N_AGENTS * SAMPLE)
        W = W.transpose(2, 1, 0).reshape(N_AGENTS * N_AGENTS, bs)
    return W


def kernel(states, actions, agent_qs, max_filter, target,
           hw1_w1, hw1_b1, hw1_w2, hw1_b2,
           hwf_w1, hwf_b1, hwf_w2, hwf_b2,
           hb1_w, hb1_b, v_w1, v_b1, v_w2, v_b2):
    B0, T0 = states.shape[0], states.shape[1]
    bs = B0 * T0
    Wq = _coalition_weights(bs)

    rs = states.reshape(bs, STATE_DIM)
    ra = actions.reshape(bs, N_AGENTS * N_ACTIONS).astype(jnp.float32)
    aq = agent_qs.reshape(bs, N_AGENTS)

    row = lambda i: (i, 0)
    col = lambda i: (0, i)
    rep = lambda i: (0, 0)
    grid = (bs // BLK,)
    out = pl.pallas_call(
        _mixer_kernel,
        grid=grid,
        in_specs=[
            pl.BlockSpec((BLK, STATE_DIM), row),
            pl.BlockSpec((BLK, N_AGENTS * N_ACTIONS), row),
            pl.BlockSpec((N_AGENTS * N_AGENTS, BLK), col),
            pl.BlockSpec((BLK, N_AGENTS), row),
            pl.BlockSpec((STATE_DIM, 256), rep),
            pl.BlockSpec((256, 1), rep),
            pl.BlockSpec((256, 2 * N_ACTIONS * EMBED), rep),
            pl.BlockSpec((2 * N_ACTIONS * EMBED, 1), rep),
            pl.BlockSpec((STATE_DIM, 256), rep),
            pl.BlockSpec((256, 1), rep),
            pl.BlockSpec((256, EMBED), rep),
            pl.BlockSpec((EMBED, 1), rep),
            pl.BlockSpec((STATE_DIM, EMBED), rep),
            pl.BlockSpec((EMBED, 1), rep),
            pl.BlockSpec((STATE_DIM, EMBED), rep),
            pl.BlockSpec((EMBED, 1), rep),
            pl.BlockSpec((EMBED, 1), rep),
            pl.BlockSpec((1, 1), rep),
        ],
        out_specs=[
            pl.BlockSpec((BLK, N_AGENTS), row),
            pl.BlockSpec((BLK, 1), row),
        ],
        out_shape=[
            jax.ShapeDtypeStruct((bs, N_AGENTS), jnp.float32),
            jax.ShapeDtypeStruct((bs, 1), jnp.float32),
        ],
    )(
        rs, ra, Wq, aq,
        hw1_w1, hw1_b1.reshape(-1, 1), hw1_w2, hw1_b2.reshape(-1, 1),
        hwf_w1, hwf_b1.reshape(-1, 1), hwf_w2, hwf_b2.reshape(-1, 1),
        hb1_w, hb1_b.reshape(-1, 1), v_w1, v_b1.reshape(-1, 1),
        v_w2, v_b2.reshape(1, 1),
    )
    w_est = out[0].reshape(B0, T0, N_AGENTS)
    q_tot = out[1].reshape(B0, T0, 1)
    q_tot = jnp.where(target != 0,
                      jnp.sum(agent_qs, axis=2, keepdims=True), q_tot)
    return q_tot, w_est
